# Initial kernel scaffold; baseline (speedup 1.0000x reference)
import numpy as np
import jax
import jax.numpy as jnp
from jax import lax
from jax.experimental import pallas as pl
from jax.experimental.pallas import tpu as pltpu

_INV_BN = np.float32(1.0) / np.sqrt(np.float32(1.0 + 1e-5))


def _flat_iota(shape):
    return (lax.broadcasted_iota(jnp.int32, shape, 0) * shape[1]
            + lax.broadcasted_iota(jnp.int32, shape, 1))


def _fps_call(pxg, pyg, pzg, K):
    R = pxg.shape[0]
    N = R * 128
    KR = max(K // 128, 1)

    def body(px_ref, py_ref, pz_ref, ox_ref, oy_ref, oz_ref, d_ref):
        px = px_ref[...]
        py = py_ref[...]
        pz = pz_ref[...]
        fi = _flat_iota((R, 128))
        ki = _flat_iota((KR, 128))

        def point_at(j):
            m = fi == j
            return (jnp.sum(jnp.where(m, px, 0.0)),
                    jnp.sum(jnp.where(m, py, 0.0)),
                    jnp.sum(jnp.where(m, pz, 0.0)))

        def dist_to(xj, yj, zj):
            dx = px - xj
            dy = py - yj
            dz = pz - zj
            return jnp.sqrt(dx * dx + dy * dy + dz * dz)

        x0, y0, z0 = point_at(jnp.int32(0))
        d_ref[...] = dist_to(x0, y0, z0)
        ox_ref[...] = jnp.where(ki == 0, x0, 0.0)
        oy_ref[...] = jnp.where(ki == 0, y0, 0.0)
        oz_ref[...] = jnp.where(ki == 0, z0, 0.0)

        def step(i, carry):
            d = d_ref[...]
            mx = jnp.max(d)
            j = jnp.min(jnp.where(d == mx, fi, jnp.int32(N)))
            xj, yj, zj = point_at(j)
            d_ref[...] = jnp.minimum(d, dist_to(xj, yj, zj))
            sel = ki == i
            ox_ref[...] = jnp.where(sel, xj, ox_ref[...])
            oy_ref[...] = jnp.where(sel, yj, oy_ref[...])
            oz_ref[...] = jnp.where(sel, zj, oz_ref[...])
            return carry

        lax.fori_loop(1, K, step, 0)

    return pl.pallas_call(
        body,
        out_shape=[jax.ShapeDtypeStruct((KR, 128), jnp.float32)] * 3,
        scratch_shapes=[pltpu.VMEM((R, 128), jnp.float32)],
    )(pxg, pyg, pzg)


def _sa_call(Xin, W1t, b1, W2t, b2, W3t, b3, cx, cy, cz, pxr, pyr, pzr,
             radius, CB, PB):
    N, Din = Xin.shape
    C = cx.shape[0]
    H1 = W1t.shape[1]
    H2 = W2t.shape[1]
    H3 = W3t.shape[1]
    r32 = np.float32(radius)

    def body(x_ref, w1_ref, b1_ref, w2_ref, b2_ref, w3_ref, b3_ref,
             cx_ref, cy_ref, cz_ref, px_ref, py_ref, pz_ref, o_ref):
        j = pl.program_id(1)
        W1 = w1_ref[...]
        base = jnp.dot(x_ref[...], W1,
                       preferred_element_type=jnp.float32) + b1_ref[...]
        cxb = cx_ref[...]
        cyb = cy_ref[...]
        czb = cz_ref[...]
        coff = cxb * W1[0:1] + cyb * W1[1:2] + czb * W1[2:3]
        h1 = jnp.maximum(base[None, :, :] - coff[:, None, :], 0.0)
        h1 = h1.reshape(CB * PB, H1)
        h2 = jnp.maximum(
            jnp.dot(h1, w2_ref[...], preferred_element_type=jnp.float32)
            + b2_ref[...], 0.0)
        h3 = jnp.maximum(
            jnp.dot(h2, w3_ref[...], preferred_element_type=jnp.float32)
            + b3_ref[...], 0.0)
        dx = cxb - px_ref[...]
        dy = cyb - py_ref[...]
        dz = czb - pz_ref[...]
        mask = jnp.sqrt(dx * dx + dy * dy + dz * dz) < r32
        h3 = h3.reshape(CB, PB, H3)
        part = jnp.max(jnp.where(mask[:, :, None], h3, -jnp.inf), axis=1)

        @pl.when(j == 0)
        def _():
            o_ref[...] = part

        @pl.when(j > 0)
        def _():
            o_ref[...] = jnp.maximum(o_ref[...], part)

    return pl.pallas_call(
        body,
        grid=(C // CB, N // PB),
        in_specs=[
            pl.BlockSpec((PB, Din), lambda i, j: (j, 0)),
            pl.BlockSpec((Din, H1), lambda i, j: (0, 0)),
            pl.BlockSpec((1, H1), lambda i, j: (0, 0)),
            pl.BlockSpec((H1, H2), lambda i, j: (0, 0)),
            pl.BlockSpec((1, H2), lambda i, j: (0, 0)),
            pl.BlockSpec((H2, H3), lambda i, j: (0, 0)),
            pl.BlockSpec((1, H3), lambda i, j: (0, 0)),
            pl.BlockSpec((CB, 1), lambda i, j: (i, 0)),
            pl.BlockSpec((CB, 1), lambda i, j: (i, 0)),
            pl.BlockSpec((CB, 1), lambda i, j: (i, 0)),
            pl.BlockSpec((1, PB), lambda i, j: (0, j)),
            pl.BlockSpec((1, PB), lambda i, j: (0, j)),
            pl.BlockSpec((1, PB), lambda i, j: (0, j)),
        ],
        out_specs=pl.BlockSpec((CB, H3), lambda i, j: (i, 0)),
        out_shape=jax.ShapeDtypeStruct((C, H3), jnp.float32),
    )(Xin, W1t, b1, W2t, b2, W3t, b3, cx, cy, cz, pxr, pyr, pzr)


def _tail_call(c2, f2, w1c, w1f, b1, w2, b2, w3, b3,
               wa, wb, bb, g0, be0, w2f, b2f, g1, be1):
    C2, F2 = f2.shape
    OUT = w2f.shape[1]

    def body(c_ref, f_ref, w1c_ref, w1f_ref, b1_ref, w2_ref, b2_ref,
             w3_ref, b3_ref, wa_ref, wb_ref, bb_ref, g0_ref, be0_ref,
             w2f_ref, b2f_ref, g1_ref, be1_ref, o_ref):
        c = c_ref[...]
        d = c - c[0:1, :]
        f2v = f_ref[...]
        h = jnp.maximum(
            jnp.dot(d, w1c_ref[...], preferred_element_type=jnp.float32)
            + jnp.dot(f2v, w1f_ref[...], preferred_element_type=jnp.float32)
            + b1_ref[...], 0.0)
        h = jnp.maximum(
            jnp.dot(h, w2_ref[...], preferred_element_type=jnp.float32)
            + b2_ref[...], 0.0)
        h = jnp.maximum(
            jnp.dot(h, w3_ref[...], preferred_element_type=jnp.float32)
            + b3_ref[...], 0.0)
        fm = jnp.max(h, axis=0, keepdims=True)
        kmw = jnp.dot(fm, wb_ref[...], preferred_element_type=jnp.float32)
        y = (jnp.dot(f2v, wa_ref[...], preferred_element_type=jnp.float32)
             + kmw + bb_ref[...])
        y = jnp.maximum(g0_ref[...] * y * _INV_BN + be0_ref[...], 0.0)
        y = jnp.dot(y, w2f_ref[...], preferred_element_type=jnp.float32) \
            + b2f_ref[...]
        y = jnp.maximum(g1_ref[...] * y * _INV_BN + be1_ref[...], 0.0)
        o_ref[...] = y

    return pl.pallas_call(
        body,
        out_shape=jax.ShapeDtypeStruct((C2, OUT), jnp.float32),
    )(c2, f2, w1c, w1f, b1, w2, b2, w3, b3,
      wa, wb, bb, g0, be0, w2f, b2f, g1, be1)


def _fp_call(qcols, prows, from_f, f_prev, layer_arrays, bn_flags, QB):
    Q = qcols[0].shape[0]
    P, F = from_f.shape
    Dprev = f_prev.shape[1]
    OUT = layer_arrays[-1][0].shape[1]

    flat = []
    for arrs in layer_arrays:
        flat.extend(arrs)
    n_flat = len(flat)

    def body(*refs):
        qx_ref, qy_ref, qz_ref, px_ref, py_ref, pz_ref, ff_ref, fp_ref = \
            refs[:8]
        lrefs = list(refs[8:8 + n_flat])
        o_ref = refs[8 + n_flat]
        dx = px_ref[...] - qx_ref[...]
        dy = py_ref[...] - qy_ref[...]
        dz = pz_ref[...] - qz_ref[...]
        sq = dx * dx + dy * dy + dz * dz
        zero = sq == 0.0
        dmat = jnp.where(zero, 0.0, jnp.sqrt(jnp.where(zero, 1.0, sq)))
        col = lax.broadcasted_iota(jnp.int32, (QB, P), 1)
        dwork = dmat
        dks, iks = [], []
        for _ in range(3):
            mk = jnp.min(dwork, axis=1, keepdims=True)
            ik = jnp.min(jnp.where(dwork == mk, col, jnp.int32(P)),
                         axis=1, keepdims=True)
            dks.append(mk)
            iks.append(ik)
            dwork = jnp.where(col == ik, jnp.float32(jnp.inf), dwork)
        iz = [dk == 0.0 for dk in dks]
        any_zero = iz[0] | iz[1] | iz[2]
        raws = []
        for z, dk in zip(iz, dks):
            safe = jnp.where(z, 1.0, dk)
            raws.append(1.0 / (safe * safe))
        s = raws[0] + raws[1] + raws[2]
        Wc = jnp.zeros((QB, P), jnp.float32)
        for k in range(3):
            wk = jnp.where(any_zero, iz[k].astype(jnp.float32), raws[k] / s)
            Wc = Wc + jnp.where(col == iks[k], wk, 0.0)
        km = jnp.dot(Wc, ff_ref[...], preferred_element_type=jnp.float32)
        wp = lrefs[0][...]
        wk_ = lrefs[1][...]
        b = lrefs[2][...]
        x = (jnp.dot(fp_ref[...], wp, preferred_element_type=jnp.float32)
             + jnp.dot(km, wk_, preferred_element_type=jnp.float32) + b)
        li = 3
        if bn_flags[0]:
            x = jnp.maximum(lrefs[li][...] * x * _INV_BN
                            + lrefs[li + 1][...], 0.0)
            li += 2
        for has_bn in bn_flags[1:]:
            w = lrefs[li][...]
            b = lrefs[li + 1][...]
            x = jnp.dot(x, w, preferred_element_type=jnp.float32) + b
            li += 2
            if has_bn:
                x = jnp.maximum(lrefs[li][...] * x * _INV_BN
                                + lrefs[li + 1][...], 0.0)
                li += 2
        o_ref[...] = x

    def full2(a):
        s = a.shape
        return pl.BlockSpec(s, lambda i: (0, 0))

    in_specs = [
        pl.BlockSpec((QB, 1), lambda i: (i, 0)),
        pl.BlockSpec((QB, 1), lambda i: (i, 0)),
        pl.BlockSpec((QB, 1), lambda i: (i, 0)),
        full2(prows[0]), full2(prows[1]), full2(prows[2]),
        full2(from_f),
        pl.BlockSpec((QB, Dprev), lambda i: (i, 0)),
    ] + [full2(a) for a in flat]
    return pl.pallas_call(
        body,
        grid=(Q // QB,),
        in_specs=in_specs,
        out_specs=pl.BlockSpec((QB, OUT), lambda i: (i, 0)),
        out_shape=jax.ShapeDtypeStruct((Q, OUT), jnp.float32),
    )(*qcols, *prows, from_f, f_prev, *flat)


def kernel(coords, features, params):
    coords = coords.astype(jnp.float32)
    features = features.astype(jnp.float32)
    N = coords.shape[0]

    pxg = coords[:, 0].reshape(N // 128, 128)
    pyg = coords[:, 1].reshape(N // 128, 128)
    pzg = coords[:, 2].reshape(N // 128, 128)

    c1x, c1y, c1z = _fps_call(pxg, pyg, pzg, 512)
    c2x, c2y, c2z = _fps_call(c1x, c1y, c1z, 128)

    sa0 = params['sa'][0]
    Xin1 = jnp.concatenate([coords, features], axis=1)
    f1 = _sa_call(
        Xin1, sa0['W1'].T, sa0['b1'][None, :], sa0['W2'].T, sa0['b2'][None, :],
        sa0['W3'].T, sa0['b3'][None, :],
        c1x.reshape(512, 1), c1y.reshape(512, 1), c1z.reshape(512, 1),
        coords[:, 0].reshape(1, N), coords[:, 1].reshape(1, N),
        coords[:, 2].reshape(1, N),
        0.2, CB=8, PB=512)

    c1coords = jnp.stack(
        [c1x.reshape(-1), c1y.reshape(-1), c1z.reshape(-1)], axis=1)
    sa1 = params['sa'][1]
    Xin2 = jnp.concatenate([c1coords, f1], axis=1)
    f2 = _sa_call(
        Xin2, sa1['W1'].T, sa1['b1'][None, :], sa1['W2'].T, sa1['b2'][None, :],
        sa1['W3'].T, sa1['b3'][None, :],
        c2x.reshape(128, 1), c2y.reshape(128, 1), c2z.reshape(128, 1),
        c1x.reshape(1, 512), c1y.reshape(1, 512), c1z.reshape(1, 512),
        0.4, CB=8, PB=512)

    c2coords = jnp.stack(
        [c2x.reshape(-1), c2y.reshape(-1), c2z.reshape(-1)], axis=1)
    sa2 = params['sa'][2]
    fp0 = params['fp'][0]
    W1t = sa2['W1'].T
    fp0W0t = fp0[0]['W'].T
    g2 = _tail_call(
        c2coords, f2,
        W1t[:3], W1t[3:], sa2['b1'][None, :],
        sa2['W2'].T, sa2['b2'][None, :], sa2['W3'].T, sa2['b3'][None, :],
        fp0W0t[:256], fp0W0t[256:], fp0[0]['b'][None, :],
        fp0[0]['gamma'][None, :], fp0[0]['beta'][None, :],
        fp0[1]['W'].T, fp0[1]['b'][None, :],
        fp0[1]['gamma'][None, :], fp0[1]['beta'][None, :])

    fp1 = params['fp'][1]
    W0t = fp1[0]['W'].T
    layer_arrays1 = [
        (W0t[:128], W0t[128:], fp1[0]['b'][None, :],
         fp1[0]['gamma'][None, :], fp1[0]['beta'][None, :]),
        (fp1[1]['W'].T, fp1[1]['b'][None, :],
         fp1[1]['gamma'][None, :], fp1[1]['beta'][None, :]),
    ]
    g1 = _fp_call(
        (c1x.reshape(512, 1), c1y.reshape(512, 1), c1z.reshape(512, 1)),
        (c2x.reshape(1, 128), c2y.reshape(1, 128), c2z.reshape(1, 128)),
        g2, f1, layer_arrays1, [True, True], QB=512)

    fp2 = params['fp'][2]
    W0t2 = fp2[0]['W'].T
    layer_arrays2 = [
        (W0t2[:3], W0t2[3:], fp2[0]['b'][None, :],
         fp2[0]['gamma'][None, :], fp2[0]['beta'][None, :]),
        (fp2[1]['W'].T, fp2[1]['b'][None, :],
         fp2[1]['gamma'][None, :], fp2[1]['beta'][None, :]),
        (fp2[2]['W'].T, fp2[2]['b'][None, :]),
    ]
    out = _fp_call(
        (coords[:, 0:1], coords[:, 1:2], coords[:, 2:3]),
        (c1x.reshape(1, 512), c1y.reshape(1, 512), c1z.reshape(1, 512)),
        g1, features, layer_arrays2, [True, True, False], QB=512)
    return out


# speedup vs baseline: 3.8033x; 3.8033x over previous
import numpy as np
import jax
import jax.numpy as jnp
from jax import lax
from jax.experimental import pallas as pl
from jax.experimental.pallas import tpu as pltpu

_INV_BN = np.float32(1.0) / np.sqrt(np.float32(1.0 + 1e-5))


def _flat_iota(shape):
    return (lax.broadcasted_iota(jnp.int32, shape, 0) * shape[1]
            + lax.broadcasted_iota(jnp.int32, shape, 1))


def _fps_call(pxg, pyg, pzg, K):
    R = pxg.shape[0]
    N = R * 128
    KR = max(K // 128, 1)

    def body(px_ref, py_ref, pz_ref, ox_ref, oy_ref, oz_ref, d_ref):
        px = px_ref[...]
        py = py_ref[...]
        pz = pz_ref[...]
        fi = _flat_iota((R, 128))
        ki = _flat_iota((KR, 128))

        def point_at(j):
            m = fi == j
            return (jnp.sum(jnp.where(m, px, 0.0)),
                    jnp.sum(jnp.where(m, py, 0.0)),
                    jnp.sum(jnp.where(m, pz, 0.0)))

        def dist_to(xj, yj, zj):
            dx = px - xj
            dy = py - yj
            dz = pz - zj
            return jnp.sqrt(dx * dx + dy * dy + dz * dz)

        x0, y0, z0 = point_at(jnp.int32(0))
        d_ref[...] = dist_to(x0, y0, z0)
        ox_ref[...] = jnp.where(ki == 0, x0, 0.0)
        oy_ref[...] = jnp.where(ki == 0, y0, 0.0)
        oz_ref[...] = jnp.where(ki == 0, z0, 0.0)

        def step(i, carry):
            d = d_ref[...]
            mx = jnp.max(d)
            j = jnp.min(jnp.where(d == mx, fi, jnp.int32(N)))
            xj, yj, zj = point_at(j)
            d_ref[...] = jnp.minimum(d, dist_to(xj, yj, zj))
            sel = ki == i
            ox_ref[...] = jnp.where(sel, xj, ox_ref[...])
            oy_ref[...] = jnp.where(sel, yj, oy_ref[...])
            oz_ref[...] = jnp.where(sel, zj, oz_ref[...])
            return carry

        lax.fori_loop(1, K, step, 0)

    return pl.pallas_call(
        body,
        out_shape=[jax.ShapeDtypeStruct((KR, 128), jnp.float32)] * 3,
        scratch_shapes=[pltpu.VMEM((R, 128), jnp.float32)],
    )(pxg, pyg, pzg)


def _sa_call(Xin, W1t, b1, W2t, b2, W3t, b3, centers, pxc, pyc, pzc,
             radius, CB):
    N, Din = Xin.shape
    C = centers.shape[0]
    H1 = W1t.shape[1]
    H3 = W3t.shape[1]
    r32 = np.float32(radius)
    ninf = np.float32(-np.inf)

    def body(x_ref, w1_ref, b1_ref, w2_ref, b2_ref, w3_ref, b3_ref,
             c_ref, px_ref, py_ref, pz_ref, o_ref):
        W1 = w1_ref[...]
        base = jnp.dot(x_ref[...], W1,
                       preferred_element_type=jnp.float32) + b1_ref[...]
        px = px_ref[...]
        py = py_ref[...]
        pz = pz_ref[...]
        W2 = w2_ref[...]
        b2v = b2_ref[...]
        W3 = w3_ref[...]
        b3v = b3_ref[...]
        for c in range(CB):
            cxs = c_ref[c, 0]
            cys = c_ref[c, 1]
            czs = c_ref[c, 2]
            coff = cxs * W1[0:1] + cys * W1[1:2] + czs * W1[2:3]
            h = jnp.maximum(base - coff, 0.0)
            h = jnp.maximum(
                jnp.dot(h, W2, preferred_element_type=jnp.float32) + b2v, 0.0)
            h = jnp.maximum(
                jnp.dot(h, W3, preferred_element_type=jnp.float32) + b3v, 0.0)
            dx = cxs - px
            dy = cys - py
            dz = czs - pz
            pen = jnp.where(
                jnp.sqrt(dx * dx + dy * dy + dz * dz) < r32, 0.0, ninf)
            o_ref[c:c + 1, :] = jnp.max(h + pen, axis=0, keepdims=True)

    return pl.pallas_call(
        body,
        grid=(C // CB,),
        in_specs=[
            pl.BlockSpec((N, Din), lambda i: (0, 0)),
            pl.BlockSpec((Din, H1), lambda i: (0, 0)),
            pl.BlockSpec(b1.shape, lambda i: (0, 0)),
            pl.BlockSpec(W2t.shape, lambda i: (0, 0)),
            pl.BlockSpec(b2.shape, lambda i: (0, 0)),
            pl.BlockSpec(W3t.shape, lambda i: (0, 0)),
            pl.BlockSpec(b3.shape, lambda i: (0, 0)),
            pl.BlockSpec((CB, 3), lambda i: (i, 0),
                         memory_space=pltpu.SMEM),
            pl.BlockSpec((N, 1), lambda i: (0, 0)),
            pl.BlockSpec((N, 1), lambda i: (0, 0)),
            pl.BlockSpec((N, 1), lambda i: (0, 0)),
        ],
        out_specs=pl.BlockSpec((CB, H3), lambda i: (i, 0)),
        out_shape=jax.ShapeDtypeStruct((C, H3), jnp.float32),
    )(Xin, W1t, b1, W2t, b2, W3t, b3, centers, pxc, pyc, pzc)


def _tail_call(c2cols, c2smem, f2, w1c, w1f, b1, w2, b2, w3, b3,
               wa, wb, bb, g0, be0, w2f, b2f, g1, be1):
    C2, F2 = f2.shape
    OUT = w2f.shape[1]

    def body(cx_ref, cy_ref, cz_ref, cs_ref, f_ref, w1c_ref, w1f_ref,
             b1_ref, w2_ref, b2_ref, w3_ref, b3_ref, wa_ref, wb_ref,
             bb_ref, g0_ref, be0_ref, w2f_ref, b2f_ref, g1_ref, be1_ref,
             o_ref):
        dx = cx_ref[...] - cs_ref[0, 0]
        dy = cy_ref[...] - cs_ref[0, 1]
        dz = cz_ref[...] - cs_ref[0, 2]
        W1c = w1c_ref[...]
        dpart = dx * W1c[0:1] + dy * W1c[1:2] + dz * W1c[2:3]
        f2v = f_ref[...]
        h = jnp.maximum(
            dpart
            + jnp.dot(f2v, w1f_ref[...], preferred_element_type=jnp.float32)
            + b1_ref[...], 0.0)
        h = jnp.maximum(
            jnp.dot(h, w2_ref[...], preferred_element_type=jnp.float32)
            + b2_ref[...], 0.0)
        h = jnp.maximum(
            jnp.dot(h, w3_ref[...], preferred_element_type=jnp.float32)
            + b3_ref[...], 0.0)
        fm = jnp.max(h, axis=0, keepdims=True)
        kmw = jnp.dot(fm, wb_ref[...], preferred_element_type=jnp.float32)
        y = (jnp.dot(f2v, wa_ref[...], preferred_element_type=jnp.float32)
             + kmw + bb_ref[...])
        y = jnp.maximum(g0_ref[...] * y * _INV_BN + be0_ref[...], 0.0)
        y = jnp.dot(y, w2f_ref[...], preferred_element_type=jnp.float32) \
            + b2f_ref[...]
        y = jnp.maximum(g1_ref[...] * y * _INV_BN + be1_ref[...], 0.0)
        o_ref[...] = y

    vspec = lambda a: pl.BlockSpec(a.shape, lambda: (0,) * a.ndim)
    args = (*c2cols, c2smem, f2, w1c, w1f, b1, w2, b2, w3, b3,
            wa, wb, bb, g0, be0, w2f, b2f, g1, be1)
    in_specs = [vspec(a) for a in args]
    in_specs[3] = pl.BlockSpec(c2smem.shape, lambda: (0, 0),
                               memory_space=pltpu.SMEM)
    return pl.pallas_call(
        body,
        in_specs=in_specs,
        out_specs=pl.BlockSpec((C2, OUT), lambda: (0, 0)),
        out_shape=jax.ShapeDtypeStruct((C2, OUT), jnp.float32),
    )(*args)


def _fp_call(qcols, prows, from_f, f_prev, layer_arrays, bn_flags, QB):
    Q = qcols[0].shape[0]
    P, F = from_f.shape
    Dprev = f_prev.shape[1]
    OUT = layer_arrays[-1][0].shape[1]

    flat = []
    for arrs in layer_arrays:
        flat.extend(arrs)
    n_flat = len(flat)

    def body(*refs):
        qx_ref, qy_ref, qz_ref, px_ref, py_ref, pz_ref, ff_ref, fp_ref = \
            refs[:8]
        lrefs = list(refs[8:8 + n_flat])
        o_ref = refs[8 + n_flat]
        dx = px_ref[...] - qx_ref[...]
        dy = py_ref[...] - qy_ref[...]
        dz = pz_ref[...] - qz_ref[...]
        sq = dx * dx + dy * dy + dz * dz
        zero = sq == 0.0
        dmat = jnp.where(zero, 0.0, jnp.sqrt(jnp.where(zero, 1.0, sq)))
        col = lax.broadcasted_iota(jnp.int32, (QB, P), 1)
        dwork = dmat
        dks, iks = [], []
        for _ in range(3):
            mk = jnp.min(dwork, axis=1, keepdims=True)
            ik = jnp.min(jnp.where(dwork == mk, col, jnp.int32(P)),
                         axis=1, keepdims=True)
            dks.append(mk)
            iks.append(ik)
            dwork = jnp.where(col == ik, jnp.float32(jnp.inf), dwork)
        iz = [dk == 0.0 for dk in dks]
        any_zero = iz[0] | iz[1] | iz[2]
        raws = []
        for z, dk in zip(iz, dks):
            safe = jnp.where(z, 1.0, dk)
            raws.append(1.0 / (safe * safe))
        s = raws[0] + raws[1] + raws[2]
        Wc = jnp.zeros((QB, P), jnp.float32)
        for k in range(3):
            wk = jnp.where(any_zero, iz[k].astype(jnp.float32), raws[k] / s)
            Wc = Wc + jnp.where(col == iks[k], wk, 0.0)
        km = jnp.dot(Wc, ff_ref[...], preferred_element_type=jnp.float32)
        wp = lrefs[0][...]
        wk_ = lrefs[1][...]
        b = lrefs[2][...]
        x = (jnp.dot(fp_ref[...], wp, preferred_element_type=jnp.float32)
             + jnp.dot(km, wk_, preferred_element_type=jnp.float32) + b)
        li = 3
        if bn_flags[0]:
            x = jnp.maximum(lrefs[li][...] * x * _INV_BN
                            + lrefs[li + 1][...], 0.0)
            li += 2
        for has_bn in bn_flags[1:]:
            w = lrefs[li][...]
            b = lrefs[li + 1][...]
            x = jnp.dot(x, w, preferred_element_type=jnp.float32) + b
            li += 2
            if has_bn:
                x = jnp.maximum(lrefs[li][...] * x * _INV_BN
                                + lrefs[li + 1][...], 0.0)
                li += 2
        o_ref[...] = x

    def full2(a):
        s = a.shape
        return pl.BlockSpec(s, lambda i: (0, 0))

    in_specs = [
        pl.BlockSpec((QB, 1), lambda i: (i, 0)),
        pl.BlockSpec((QB, 1), lambda i: (i, 0)),
        pl.BlockSpec((QB, 1), lambda i: (i, 0)),
        full2(prows[0]), full2(prows[1]), full2(prows[2]),
        full2(from_f),
        pl.BlockSpec((QB, Dprev), lambda i: (i, 0)),
    ] + [full2(a) for a in flat]
    return pl.pallas_call(
        body,
        grid=(Q // QB,),
        in_specs=in_specs,
        out_specs=pl.BlockSpec((QB, OUT), lambda i: (i, 0)),
        out_shape=jax.ShapeDtypeStruct((Q, OUT), jnp.float32),
    )(*qcols, *prows, from_f, f_prev, *flat)


def kernel(coords, features, params):
    coords = coords.astype(jnp.float32)
    features = features.astype(jnp.float32)
    N = coords.shape[0]

    pxg = coords[:, 0].reshape(N // 128, 128)
    pyg = coords[:, 1].reshape(N // 128, 128)
    pzg = coords[:, 2].reshape(N // 128, 128)

    c1x, c1y, c1z = _fps_call(pxg, pyg, pzg, 512)
    c2x, c2y, c2z = _fps_call(c1x, c1y, c1z, 128)

    c1coords = jnp.stack(
        [c1x.reshape(-1), c1y.reshape(-1), c1z.reshape(-1)], axis=1)
    c2coords = jnp.stack(
        [c2x.reshape(-1), c2y.reshape(-1), c2z.reshape(-1)], axis=1)

    sa0 = params['sa'][0]
    Xin1 = jnp.concatenate([coords, features], axis=1)
    f1 = _sa_call(
        Xin1, sa0['W1'].T, sa0['b1'][None, :], sa0['W2'].T, sa0['b2'][None, :],
        sa0['W3'].T, sa0['b3'][None, :],
        c1coords,
        coords[:, 0:1], coords[:, 1:2], coords[:, 2:3],
        0.2, CB=8)

    sa1 = params['sa'][1]
    Xin2 = jnp.concatenate([c1coords, f1], axis=1)
    f2 = _sa_call(
        Xin2, sa1['W1'].T, sa1['b1'][None, :], sa1['W2'].T, sa1['b2'][None, :],
        sa1['W3'].T, sa1['b3'][None, :],
        c2coords,
        c1x.reshape(512, 1), c1y.reshape(512, 1), c1z.reshape(512, 1),
        0.4, CB=8)

    sa2 = params['sa'][2]
    fp0 = params['fp'][0]
    W1t = sa2['W1'].T
    fp0W0t = fp0[0]['W'].T
    g2 = _tail_call(
        (c2x.reshape(128, 1), c2y.reshape(128, 1), c2z.reshape(128, 1)),
        c2coords, f2,
        W1t[:3], W1t[3:], sa2['b1'][None, :],
        sa2['W2'].T, sa2['b2'][None, :], sa2['W3'].T, sa2['b3'][None, :],
        fp0W0t[:256], fp0W0t[256:], fp0[0]['b'][None, :],
        fp0[0]['gamma'][None, :], fp0[0]['beta'][None, :],
        fp0[1]['W'].T, fp0[1]['b'][None, :],
        fp0[1]['gamma'][None, :], fp0[1]['beta'][None, :])

    fp1 = params['fp'][1]
    W0t = fp1[0]['W'].T
    layer_arrays1 = [
        (W0t[:128], W0t[128:], fp1[0]['b'][None, :],
         fp1[0]['gamma'][None, :], fp1[0]['beta'][None, :]),
        (fp1[1]['W'].T, fp1[1]['b'][None, :],
         fp1[1]['gamma'][None, :], fp1[1]['beta'][None, :]),
    ]
    g1 = _fp_call(
        (c1x.reshape(512, 1), c1y.reshape(512, 1), c1z.reshape(512, 1)),
        (c2x.reshape(1, 128), c2y.reshape(1, 128), c2z.reshape(1, 128)),
        g2, f1, layer_arrays1, [True, True], QB=512)

    fp2 = params['fp'][2]
    W0t2 = fp2[0]['W'].T
    layer_arrays2 = [
        (W0t2[:3], W0t2[3:], fp2[0]['b'][None, :],
         fp2[0]['gamma'][None, :], fp2[0]['beta'][None, :]),
        (fp2[1]['W'].T, fp2[1]['b'][None, :],
         fp2[1]['gamma'][None, :], fp2[1]['beta'][None, :]),
        (fp2[2]['W'].T, fp2[2]['b'][None, :]),
    ]
    out = _fp_call(
        (coords[:, 0:1], coords[:, 1:2], coords[:, 2:3]),
        (c1x.reshape(1, 512), c1y.reshape(1, 512), c1z.reshape(1, 512)),
        g1, features, layer_arrays2, [True, True, False], QB=512)
    return out


# speedup vs baseline: 8.4652x; 2.2257x over previous
import numpy as np
import jax
import jax.numpy as jnp
from jax import lax
from jax.experimental import pallas as pl
from jax.experimental.pallas import tpu as pltpu

_INV_BN = np.float32(1.0) / np.sqrt(np.float32(1.0 + 1e-5))


def _flat_iota(shape):
    return (lax.broadcasted_iota(jnp.int32, shape, 0) * shape[1]
            + lax.broadcasted_iota(jnp.int32, shape, 1))


def _fps_call(pxg, pyg, pzg, pts_smem, K):
    R = pxg.shape[0]
    N = R * 128

    def body(px_ref, py_ref, pz_ref, ps_ref, o_ref):
        px = px_ref[...]
        py = py_ref[...]
        pz = pz_ref[...]
        fi = _flat_iota((R, 128))

        def dist_to(xj, yj, zj):
            dx = px - xj
            dy = py - yj
            dz = pz - zj
            return jnp.sqrt(dx * dx + dy * dy + dz * dz)

        x0 = ps_ref[0, 0]
        y0 = ps_ref[0, 1]
        z0 = ps_ref[0, 2]
        o_ref[0, 0] = x0
        o_ref[0, 1] = y0
        o_ref[0, 2] = z0

        def step(i, d):
            mx = jnp.max(d)
            j = jnp.min(jnp.where(d == mx, fi, jnp.int32(N)))
            xj = ps_ref[0, j * 3]
            yj = ps_ref[0, j * 3 + 1]
            zj = ps_ref[0, j * 3 + 2]
            o_ref[0, i * 3] = xj
            o_ref[0, i * 3 + 1] = yj
            o_ref[0, i * 3 + 2] = zj
            return jnp.minimum(d, dist_to(xj, yj, zj))

        lax.fori_loop(1, K, step, dist_to(x0, y0, z0))

    return pl.pallas_call(
        body,
        in_specs=[
            pl.BlockSpec((R, 128), lambda: (0, 0)),
            pl.BlockSpec((R, 128), lambda: (0, 0)),
            pl.BlockSpec((R, 128), lambda: (0, 0)),
            pl.BlockSpec((1, 3 * N), lambda: (0, 0),
                         memory_space=pltpu.SMEM),
        ],
        out_specs=pl.BlockSpec((1, 3 * K), lambda: (0, 0),
                               memory_space=pltpu.SMEM),
        out_shape=jax.ShapeDtypeStruct((1, 3 * K), jnp.float32),
    )(pxg, pyg, pzg, pts_smem)


def _sa_call(XinT, W1, b1c, w1x, w1y, w1z, W2, b2c, W3, b3c,
             centers, cxc, cyc, czc, pxr, pyr, pzr, radius, CB):
    Din, N = XinT.shape
    C = cxc.shape[0]
    H3 = W3.shape[0]
    r32 = np.float32(radius)
    ninf = np.float32(-np.inf)

    def body(x_ref, w1_ref, b1_ref, w1x_ref, w1y_ref, w1z_ref,
             w2_ref, b2_ref, w3_ref, b3_ref, c_ref,
             cx_ref, cy_ref, cz_ref, px_ref, py_ref, pz_ref, o_ref):
        baseT = jnp.dot(w1_ref[...], x_ref[...],
                        preferred_element_type=jnp.float32) + b1_ref[...]
        w1xv = w1x_ref[...]
        w1yv = w1y_ref[...]
        w1zv = w1z_ref[...]
        W2 = w2_ref[...].astype(jnp.bfloat16)
        b2v = b2_ref[...]
        W3 = w3_ref[...].astype(jnp.bfloat16)
        b3v = b3_ref[...]
        dx = cx_ref[...] - px_ref[...]
        dy = cy_ref[...] - py_ref[...]
        dz = cz_ref[...] - pz_ref[...]
        pen = jnp.where(
            jnp.sqrt(dx * dx + dy * dy + dz * dz) < r32, 0.0, ninf)
        li = lax.broadcasted_iota(jnp.int32, (H3, CB), 1)
        acc = jnp.zeros((H3, CB), jnp.float32)
        for c in range(CB):
            cxs = c_ref[c, 0]
            cys = c_ref[c, 1]
            czs = c_ref[c, 2]
            coffT = cxs * w1xv + cys * w1yv + czs * w1zv
            h = jnp.maximum(baseT - coffT, 0.0)
            h = jnp.maximum(
                jnp.dot(W2, h.astype(jnp.bfloat16),
                        preferred_element_type=jnp.float32) + b2v, 0.0)
            h = jnp.dot(W3, h.astype(jnp.bfloat16),
                        preferred_element_type=jnp.float32)
            m = jnp.max(h + pen[c:c + 1, :], axis=1, keepdims=True)
            acc = jnp.where(li == c, jnp.maximum(m + b3v, 0.0), acc)
        o_ref[0] = acc

    return pl.pallas_call(
        body,
        grid=(C // CB,),
        in_specs=[
            pl.BlockSpec((Din, N), lambda i: (0, 0)),
            pl.BlockSpec(W1.shape, lambda i: (0, 0)),
            pl.BlockSpec(b1c.shape, lambda i: (0, 0)),
            pl.BlockSpec(w1x.shape, lambda i: (0, 0)),
            pl.BlockSpec(w1y.shape, lambda i: (0, 0)),
            pl.BlockSpec(w1z.shape, lambda i: (0, 0)),
            pl.BlockSpec(W2.shape, lambda i: (0, 0)),
            pl.BlockSpec(b2c.shape, lambda i: (0, 0)),
            pl.BlockSpec(W3.shape, lambda i: (0, 0)),
            pl.BlockSpec(b3c.shape, lambda i: (0, 0)),
            pl.BlockSpec((CB, 3), lambda i: (i, 0),
                         memory_space=pltpu.SMEM),
            pl.BlockSpec((CB, 1), lambda i: (i, 0)),
            pl.BlockSpec((CB, 1), lambda i: (i, 0)),
            pl.BlockSpec((CB, 1), lambda i: (i, 0)),
            pl.BlockSpec((1, N), lambda i: (0, 0)),
            pl.BlockSpec((1, N), lambda i: (0, 0)),
            pl.BlockSpec((1, N), lambda i: (0, 0)),
        ],
        out_specs=pl.BlockSpec((1, H3, CB), lambda i: (i, 0, 0)),
        out_shape=jax.ShapeDtypeStruct((C // CB, H3, CB), jnp.float32),
    )(XinT, W1, b1c, w1x, w1y, w1z, W2, b2c, W3, b3c,
      centers, cxc, cyc, czc, pxr, pyr, pzr)


def _tail_call(c2cols, c2smem, f2, w1c, w1f, b1, w2, b2, w3, b3,
               wa, wb, bb, g0, be0, w2f, b2f, g1, be1):
    C2, F2 = f2.shape
    OUT = w2f.shape[1]

    def body(cx_ref, cy_ref, cz_ref, cs_ref, f_ref, w1c_ref, w1f_ref,
             b1_ref, w2_ref, b2_ref, w3_ref, b3_ref, wa_ref, wb_ref,
             bb_ref, g0_ref, be0_ref, w2f_ref, b2f_ref, g1_ref, be1_ref,
             o_ref):
        dx = cx_ref[...] - cs_ref[0, 0]
        dy = cy_ref[...] - cs_ref[0, 1]
        dz = cz_ref[...] - cs_ref[0, 2]
        W1c = w1c_ref[...]
        dpart = dx * W1c[0:1] + dy * W1c[1:2] + dz * W1c[2:3]
        f2v = f_ref[...]
        h = jnp.maximum(
            dpart
            + jnp.dot(f2v, w1f_ref[...], preferred_element_type=jnp.float32)
            + b1_ref[...], 0.0)
        h = jnp.maximum(
            jnp.dot(h, w2_ref[...], preferred_element_type=jnp.float32)
            + b2_ref[...], 0.0)
        h = jnp.maximum(
            jnp.dot(h, w3_ref[...], preferred_element_type=jnp.float32)
            + b3_ref[...], 0.0)
        fm = jnp.max(h, axis=0, keepdims=True)
        kmw = jnp.dot(fm, wb_ref[...], preferred_element_type=jnp.float32)
        y = (jnp.dot(f2v, wa_ref[...], preferred_element_type=jnp.float32)
             + kmw + bb_ref[...])
        y = jnp.maximum(g0_ref[...] * y * _INV_BN + be0_ref[...], 0.0)
        y = jnp.dot(y, w2f_ref[...], preferred_element_type=jnp.float32) \
            + b2f_ref[...]
        y = jnp.maximum(g1_ref[...] * y * _INV_BN + be1_ref[...], 0.0)
        o_ref[...] = y

    vspec = lambda a: pl.BlockSpec(a.shape, lambda: (0,) * a.ndim)
    args = (*c2cols, c2smem, f2, w1c, w1f, b1, w2, b2, w3, b3,
            wa, wb, bb, g0, be0, w2f, b2f, g1, be1)
    in_specs = [vspec(a) for a in args]
    in_specs[3] = pl.BlockSpec(c2smem.shape, lambda: (0, 0),
                               memory_space=pltpu.SMEM)
    return pl.pallas_call(
        body,
        in_specs=in_specs,
        out_specs=pl.BlockSpec((C2, OUT), lambda: (0, 0)),
        out_shape=jax.ShapeDtypeStruct((C2, OUT), jnp.float32),
    )(*args)


def _fp_call(qcols, prows, from_f, f_prev, layer_arrays, bn_flags, QB):
    Q = qcols[0].shape[0]
    P, F = from_f.shape
    Dprev = f_prev.shape[1]
    OUT = layer_arrays[-1][0].shape[1]

    flat = []
    for arrs in layer_arrays:
        flat.extend(arrs)
    n_flat = len(flat)

    def body(*refs):
        qx_ref, qy_ref, qz_ref, px_ref, py_ref, pz_ref, ff_ref, fp_ref = \
            refs[:8]
        lrefs = list(refs[8:8 + n_flat])
        o_ref = refs[8 + n_flat]
        dx = px_ref[...] - qx_ref[...]
        dy = py_ref[...] - qy_ref[...]
        dz = pz_ref[...] - qz_ref[...]
        sq = dx * dx + dy * dy + dz * dz
        zero = sq == 0.0
        dmat = jnp.where(zero, 0.0, jnp.sqrt(jnp.where(zero, 1.0, sq)))
        col = lax.broadcasted_iota(jnp.int32, (QB, P), 1)
        dwork = dmat
        dks, iks = [], []
        for _ in range(3):
            mk = jnp.min(dwork, axis=1, keepdims=True)
            ik = jnp.min(jnp.where(dwork == mk, col, jnp.int32(P)),
                         axis=1, keepdims=True)
            dks.append(mk)
            iks.append(ik)
            dwork = jnp.where(col == ik, jnp.float32(jnp.inf), dwork)
        iz = [dk == 0.0 for dk in dks]
        any_zero = iz[0] | iz[1] | iz[2]
        raws = []
        for z, dk in zip(iz, dks):
            safe = jnp.where(z, 1.0, dk)
            raws.append(1.0 / (safe * safe))
        s = raws[0] + raws[1] + raws[2]
        Wc = jnp.zeros((QB, P), jnp.float32)
        for k in range(3):
            wk = jnp.where(any_zero, iz[k].astype(jnp.float32), raws[k] / s)
            Wc = Wc + jnp.where(col == iks[k], wk, 0.0)
        km = jnp.dot(Wc, ff_ref[...], preferred_element_type=jnp.float32)
        wp = lrefs[0][...]
        wk_ = lrefs[1][...]
        b = lrefs[2][...]
        x = (jnp.dot(fp_ref[...], wp, preferred_element_type=jnp.float32)
             + jnp.dot(km, wk_, preferred_element_type=jnp.float32) + b)
        li = 3
        if bn_flags[0]:
            x = jnp.maximum(lrefs[li][...] * x * _INV_BN
                            + lrefs[li + 1][...], 0.0)
            li += 2
        for has_bn in bn_flags[1:]:
            w = lrefs[li][...]
            b = lrefs[li + 1][...]
            x = jnp.dot(x, w, preferred_element_type=jnp.float32) + b
            li += 2
            if has_bn:
                x = jnp.maximum(lrefs[li][...] * x * _INV_BN
                                + lrefs[li + 1][...], 0.0)
                li += 2
        o_ref[...] = x

    def full2(a):
        s = a.shape
        return pl.BlockSpec(s, lambda i: (0, 0))

    in_specs = [
        pl.BlockSpec((QB, 1), lambda i: (i, 0)),
        pl.BlockSpec((QB, 1), lambda i: (i, 0)),
        pl.BlockSpec((QB, 1), lambda i: (i, 0)),
        full2(prows[0]), full2(prows[1]), full2(prows[2]),
        full2(from_f),
        pl.BlockSpec((QB, Dprev), lambda i: (i, 0)),
    ] + [full2(a) for a in flat]
    return pl.pallas_call(
        body,
        grid=(Q // QB,),
        in_specs=in_specs,
        out_specs=pl.BlockSpec((QB, OUT), lambda i: (i, 0)),
        out_shape=jax.ShapeDtypeStruct((Q, OUT), jnp.float32),
    )(*qcols, *prows, from_f, f_prev, *flat)


def kernel(coords, features, params):
    coords = coords.astype(jnp.float32)
    features = features.astype(jnp.float32)
    N = coords.shape[0]

    pxg = coords[:, 0].reshape(N // 128, 128)
    pyg = coords[:, 1].reshape(N // 128, 128)
    pzg = coords[:, 2].reshape(N // 128, 128)

    c1coords = _fps_call(pxg, pyg, pzg, coords.reshape(1, -1),
                         512).reshape(512, 3)
    c1x = c1coords[:, 0].reshape(4, 128)
    c1y = c1coords[:, 1].reshape(4, 128)
    c1z = c1coords[:, 2].reshape(4, 128)
    c2coords = _fps_call(c1x, c1y, c1z, c1coords.reshape(1, -1),
                         128).reshape(128, 3)
    c2x = c2coords[:, 0].reshape(1, 128)
    c2y = c2coords[:, 1].reshape(1, 128)
    c2z = c2coords[:, 2].reshape(1, 128)

    def _unblock(o):
        return jnp.transpose(o, (1, 0, 2)).reshape(o.shape[1], -1)

    sa0 = params['sa'][0]
    XinT1 = jnp.concatenate([coords.T, features.T], axis=0)
    f1T = _sa_call(
        XinT1, sa0['W1'], sa0['b1'][:, None],
        sa0['W1'][:, 0:1], sa0['W1'][:, 1:2], sa0['W1'][:, 2:3],
        sa0['W2'], sa0['b2'][:, None], sa0['W3'], sa0['b3'][:, None],
        c1coords,
        c1x.reshape(512, 1), c1y.reshape(512, 1), c1z.reshape(512, 1),
        coords[:, 0].reshape(1, N), coords[:, 1].reshape(1, N),
        coords[:, 2].reshape(1, N),
        0.2, CB=8)
    f1T = _unblock(f1T)
    f1 = f1T.T

    sa1 = params['sa'][1]
    c1coordsT = jnp.stack(
        [c1x.reshape(-1), c1y.reshape(-1), c1z.reshape(-1)], axis=0)
    XinT2 = jnp.concatenate([c1coordsT, f1T], axis=0)
    f2T = _sa_call(
        XinT2, sa1['W1'], sa1['b1'][:, None],
        sa1['W1'][:, 0:1], sa1['W1'][:, 1:2], sa1['W1'][:, 2:3],
        sa1['W2'], sa1['b2'][:, None], sa1['W3'], sa1['b3'][:, None],
        c2coords,
        c2x.reshape(128, 1), c2y.reshape(128, 1), c2z.reshape(128, 1),
        c1x.reshape(1, 512), c1y.reshape(1, 512), c1z.reshape(1, 512),
        0.4, CB=8)
    f2 = _unblock(f2T).T

    sa2 = params['sa'][2]
    fp0 = params['fp'][0]
    W1t = sa2['W1'].T
    fp0W0t = fp0[0]['W'].T
    g2 = _tail_call(
        (c2x.reshape(128, 1), c2y.reshape(128, 1), c2z.reshape(128, 1)),
        c2coords, f2,
        W1t[:3], W1t[3:], sa2['b1'][None, :],
        sa2['W2'].T, sa2['b2'][None, :], sa2['W3'].T, sa2['b3'][None, :],
        fp0W0t[:256], fp0W0t[256:], fp0[0]['b'][None, :],
        fp0[0]['gamma'][None, :], fp0[0]['beta'][None, :],
        fp0[1]['W'].T, fp0[1]['b'][None, :],
        fp0[1]['gamma'][None, :], fp0[1]['beta'][None, :])

    fp1 = params['fp'][1]
    W0t = fp1[0]['W'].T
    layer_arrays1 = [
        (W0t[:128], W0t[128:], fp1[0]['b'][None, :],
         fp1[0]['gamma'][None, :], fp1[0]['beta'][None, :]),
        (fp1[1]['W'].T, fp1[1]['b'][None, :],
         fp1[1]['gamma'][None, :], fp1[1]['beta'][None, :]),
    ]
    g1 = _fp_call(
        (c1x.reshape(512, 1), c1y.reshape(512, 1), c1z.reshape(512, 1)),
        (c2x.reshape(1, 128), c2y.reshape(1, 128), c2z.reshape(1, 128)),
        g2, f1, layer_arrays1, [True, True], QB=512)

    fp2 = params['fp'][2]
    W0t2 = fp2[0]['W'].T
    layer_arrays2 = [
        (W0t2[:3], W0t2[3:], fp2[0]['b'][None, :],
         fp2[0]['gamma'][None, :], fp2[0]['beta'][None, :]),
        (fp2[1]['W'].T, fp2[1]['b'][None, :],
         fp2[1]['gamma'][None, :], fp2[1]['beta'][None, :]),
        (fp2[2]['W'].T, fp2[2]['b'][None, :]),
    ]
    out = _fp_call(
        (coords[:, 0:1], coords[:, 1:2], coords[:, 2:3]),
        (c1x.reshape(1, 512), c1y.reshape(1, 512), c1z.reshape(1, 512)),
        g1, features, layer_arrays2, [True, True, False], QB=512)
    return out
